# baseline (speedup 1.0000x reference)
import jax
import jax.numpy as jnp
from jax.experimental import pallas as pl
from jax.experimental.pallas import tpu as pltpu

NEG = -1e30
_ITERS = 5

_XT = 512
_AT = 256
_CH = 1024
_RT = 1024

_N_X = 8
_N_A = 16
_N_R = 4
_PER = _CH // _AT

_S1 = 1 + _N_X
_SUP3 = _S1 + _N_A
_L3 = _SUP3 + 1
_STEPS = _L3 + _N_R


def _bjorck(w):
    for _ in range(_ITERS):
        wtw = jax.lax.dot_general(w, w, (((0,), (0,)), ((), ())),
                                  preferred_element_type=jnp.float32)
        w = 1.5 * w - 0.5 * jnp.dot(w, wtw,
                                    preferred_element_type=jnp.float32)
    return w


def _mega_kernel(x_ref, a_ref, w0_ref, w1_ref, w2_ref,
                 b0_ref, b1_ref, b2_ref, g0_ref, m0_ref, v0_ref, be0_ref,
                 g1_ref, m1_ref, v1_ref, be1_ref,
                 o_ref,
                 wf0_s, wf1_s, wf2_s, bf0_s, bf1_s, a_s, s1_s, h1c_s, h2a_s,
                 s3_s):
    s = pl.program_id(0)
    bf16 = jnp.bfloat16

    @pl.when(s == 0)
    def _init():
        sc0 = g0_ref[...] * jax.lax.rsqrt(v0_ref[...] + 1e-5)
        wf0_s[...] = (_bjorck(w0_ref[...]) * sc0).astype(jnp.bfloat16)
        bf0_s[...] = (b0_ref[...] - m0_ref[...]) * sc0 + be0_ref[...]
        sc1 = g1_ref[...] * jax.lax.rsqrt(v1_ref[...] + 1e-5)
        bf1_s[...] = (b1_ref[...] - m1_ref[...]) * sc1 + be1_ref[...]
        h2a_s[...] = jnp.zeros_like(h2a_s)

    @pl.when((s >= 1) & (s < _S1))
    def _sup1():
        r = (s - 1) * _XT
        s1_s[pl.ds(r, _XT), :] = jnp.dot(
            x_ref[...].astype(bf16), wf0_s[...],
            preferred_element_type=jnp.float32).astype(bf16)

    @pl.when(s == _S1)
    def _bj1():
        sc1 = g1_ref[...] * jax.lax.rsqrt(v1_ref[...] + 1e-5)
        wf1_s[...] = (_bjorck(w1_ref[...]) * sc1).astype(bf16)

    @pl.when(s == _S1 + 1)
    def _bj2():
        wf2_s[...] = _bjorck(w2_ref[...]).astype(bf16)

    @pl.when((s >= _S1) & (s < _SUP3))
    def _stream():
        r = (s - _S1) * _AT
        a = a_ref[...].astype(bf16)
        a_s[pl.ds(r, _AT), :] = a
        rl = ((s - _S1) % _PER) * _AT
        h1c_s[pl.ds(rl, _AT), :] = jnp.maximum(
            jnp.dot(a, s1_s[...], preferred_element_type=jnp.float32)
            + bf0_s[...], 0.0).astype(bf16)

    @pl.when((s >= _S1) & (s < _SUP3) & ((s - _S1) % _PER == _PER - 1))
    def _l2chunk():
        c = ((s - _S1) // _PER) * _CH
        s2c = jnp.dot(h1c_s[...], wf1_s[...],
                      preferred_element_type=jnp.float32).astype(bf16)
        upd = jax.lax.dot_general(
            a_s[pl.ds(c, _CH), :], s2c, (((0,), (0,)), ((), ())),
            preferred_element_type=jnp.float32)
        h2a_s[...] = (h2a_s[...].astype(jnp.float32) + upd).astype(bf16)

    @pl.when(s == _SUP3)
    def _sup3():
        s1_s[...] = jnp.maximum(
            h2a_s[...].astype(jnp.float32) + bf1_s[...], 0.0).astype(bf16)
        s3_s[...] = jnp.dot(s1_s[...], wf2_s[...],
                            preferred_element_type=jnp.float32).astype(bf16)

    @pl.when(s >= _L3)
    def _layer3():
        r = (s - _L3) * _RT
        acc = jnp.dot(a_s[pl.ds(r, _RT), :], s3_s[...],
                      preferred_element_type=jnp.float32)
        h = acc + b2_ref[...]
        m = jnp.max(h, axis=-1, keepdims=True)
        z = h - m
        lse = jnp.log(jnp.sum(jnp.exp(z), axis=-1, keepdims=True))
        o_ref[...] = (z - lse)[:, :64]


def kernel(x, adj, w0, b0, w1, b1, w2, b2,
           g0, be0, m0, v0, g1, be1, m1, v1):
    f32 = jnp.float32
    n = x.shape[0]
    nfeat, nhid = w0.shape
    nclass = w2.shape[1]
    ncls_p = 128

    b2p = jnp.pad(b2, (0, ncls_p - nclass),
                  constant_values=NEG).reshape(1, ncls_p).astype(f32)
    w2p = jnp.pad(w2, ((0, 0), (0, ncls_p - nclass)))
    v = lambda a: a.reshape(1, nhid)

    rst = lambda i: (0, 0)
    out = pl.pallas_call(
        _mega_kernel,
        out_shape=jax.ShapeDtypeStruct((n, nclass), f32),
        grid=(_STEPS,),
        in_specs=[
            pl.BlockSpec((_XT, nfeat),
                         lambda i: (jnp.clip(i - 1, 0, _N_X - 1), 0)),
            pl.BlockSpec((_AT, n),
                         lambda i: (jnp.clip(i - _S1, 0, _N_A - 1), 0)),
            pl.BlockSpec((nfeat, nhid), rst),
            pl.BlockSpec((nhid, nhid), rst),
            pl.BlockSpec((nhid, ncls_p), rst),
        ] + [pl.BlockSpec((1, nhid), rst)] * 2
          + [pl.BlockSpec((1, ncls_p), rst)]
          + [pl.BlockSpec((1, nhid), rst)] * 8,
        out_specs=pl.BlockSpec((_RT, nclass),
                               lambda i: (jnp.clip(i - _L3, 0, _N_R - 1), 0)),
        scratch_shapes=[
            pltpu.VMEM((nfeat, nhid), jnp.bfloat16),
            pltpu.VMEM((nhid, nhid), jnp.bfloat16),
            pltpu.VMEM((nhid, ncls_p), jnp.bfloat16),
            pltpu.VMEM((1, nhid), f32),
            pltpu.VMEM((1, nhid), f32),
            pltpu.VMEM((n, n), jnp.bfloat16),
            pltpu.VMEM((n, nhid), jnp.bfloat16),
            pltpu.VMEM((_CH, nhid), jnp.bfloat16),
            pltpu.VMEM((n, nhid), jnp.bfloat16),
            pltpu.VMEM((n, ncls_p), jnp.bfloat16),
        ],
        compiler_params=pltpu.CompilerParams(
            dimension_semantics=("arbitrary",)),
    )(x, adj, w0, w1, w2p, v(b0), v(b1), b2p,
      v(g0), v(m0), v(v0), v(be0), v(g1), v(m1), v(v1), v(be1))
    return out


# speedup vs baseline: 1.0659x; 1.0659x over previous
import jax
import jax.numpy as jnp
from jax.experimental import pallas as pl
from jax.experimental.pallas import tpu as pltpu

NEG = -1e30
_ITERS = 5

_XT = 1024
_AT = 256
_RT = 1024

_N_X = 4
_N_A = 16
_N_R = 4

_S1 = 1 + _N_X
_SUP2 = _S1 + _N_A
_L2 = _SUP2 + 1
_SUP3 = _L2 + _N_R
_L3 = _SUP3 + 1
_STEPS = _L3 + _N_R


def _bjorck(w):
    for _ in range(_ITERS):
        wtw = jax.lax.dot_general(w, w, (((0,), (0,)), ((), ())),
                                  preferred_element_type=jnp.float32)
        w = 1.5 * w - 0.5 * jnp.dot(w, wtw,
                                    preferred_element_type=jnp.float32)
    return w


def _mega_kernel(x_ref, a_ref, w0_ref, w1_ref, w2_ref,
                 b0_ref, b1_ref, b2_ref, g0_ref, m0_ref, v0_ref, be0_ref,
                 g1_ref, m1_ref, v1_ref, be1_ref,
                 o_ref,
                 wf0_s, wf1_s, wf2_s, bf0_s, bf1_s, a_s, s1_s, h1_s, h2_s):
    s = pl.program_id(0)
    bf16 = jnp.bfloat16

    @pl.when(s == 0)
    def _init():
        sc0 = g0_ref[...] * jax.lax.rsqrt(v0_ref[...] + 1e-5)
        wf0_s[...] = _bjorck(w0_ref[...]) * sc0
        bf0_s[...] = (b0_ref[...] - m0_ref[...]) * sc0 + be0_ref[...]
        sc1 = g1_ref[...] * jax.lax.rsqrt(v1_ref[...] + 1e-5)
        bf1_s[...] = (b1_ref[...] - m1_ref[...]) * sc1 + be1_ref[...]

    @pl.when((s >= 1) & (s < _S1))
    def _sup1():
        r = (s - 1) * _XT
        s1_s[pl.ds(r, _XT), :] = jnp.dot(
            x_ref[...], wf0_s[...],
            preferred_element_type=jnp.float32).astype(bf16)

    @pl.when(s == _S1 + 1)
    def _bj1():
        sc1 = g1_ref[...] * jax.lax.rsqrt(v1_ref[...] + 1e-5)
        wf1_s[...] = (_bjorck(w1_ref[...]) * sc1).astype(bf16)

    @pl.when(s == _S1 + 2)
    def _bj2():
        wf2_s[...] = _bjorck(w2_ref[...]).astype(bf16)

    @pl.when((s >= _S1) & (s < _SUP2))
    def _stream():
        r = (s - _S1) * _AT
        a = a_ref[...].astype(bf16)
        a_s[pl.ds(r, _AT), :] = a
        h1_s[pl.ds(r, _AT), :] = jnp.maximum(
            jnp.dot(a, s1_s[...], preferred_element_type=jnp.float32)
            + bf0_s[...], 0.0).astype(bf16)

    @pl.when(s == _SUP2)
    def _sup2():
        s1_s[...] = jnp.dot(h1_s[...], wf1_s[...],
                            preferred_element_type=jnp.float32).astype(bf16)

    @pl.when((s >= _L2) & (s < _SUP3))
    def _layer2():
        r = (s - _L2) * _RT
        acc = jnp.dot(a_s[pl.ds(r, _RT), :], s1_s[...],
                      preferred_element_type=jnp.float32)
        h2_s[pl.ds(r, _RT), :] = jnp.maximum(
            acc + bf1_s[...], 0.0).astype(bf16)

    @pl.when(s == _SUP3)
    def _sup3():
        h1_s[:, :128] = jnp.dot(h2_s[...], wf2_s[...],
                                preferred_element_type=jnp.float32
                                ).astype(bf16)

    @pl.when(s >= _L3)
    def _layer3():
        r = (s - _L3) * _RT
        acc = jnp.dot(a_s[pl.ds(r, _RT), :], h1_s[:, :128],
                      preferred_element_type=jnp.float32)
        h = acc + b2_ref[...]
        m = jnp.max(h, axis=-1, keepdims=True)
        z = h - m
        lse = jnp.log(jnp.sum(jnp.exp(z), axis=-1, keepdims=True))
        o_ref[...] = (z - lse)[:, :64]


def kernel(x, adj, w0, b0, w1, b1, w2, b2,
           g0, be0, m0, v0, g1, be1, m1, v1):
    f32 = jnp.float32
    n = x.shape[0]
    nfeat, nhid = w0.shape
    nclass = w2.shape[1]
    ncls_p = 128

    b2p = jnp.pad(b2, (0, ncls_p - nclass),
                  constant_values=NEG).reshape(1, ncls_p).astype(f32)
    w2p = jnp.pad(w2, ((0, 0), (0, ncls_p - nclass)))
    v = lambda a: a.reshape(1, nhid)

    rst = lambda i: (0, 0)
    out = pl.pallas_call(
        _mega_kernel,
        out_shape=jax.ShapeDtypeStruct((n, nclass), f32),
        grid=(_STEPS,),
        in_specs=[
            pl.BlockSpec((_XT, nfeat),
                         lambda i: (jnp.clip(i - 1, 0, _N_X - 1), 0)),
            pl.BlockSpec((_AT, n),
                         lambda i: (jnp.clip(i - _S1, 0, _N_A - 1), 0)),
            pl.BlockSpec((nfeat, nhid), rst),
            pl.BlockSpec((nhid, nhid), rst),
            pl.BlockSpec((nhid, ncls_p), rst),
        ] + [pl.BlockSpec((1, nhid), rst)] * 2
          + [pl.BlockSpec((1, ncls_p), rst)]
          + [pl.BlockSpec((1, nhid), rst)] * 8,
        out_specs=pl.BlockSpec((_RT, nclass),
                               lambda i: (jnp.clip(i - _L3, 0, _N_R - 1), 0)),
        scratch_shapes=[
            pltpu.VMEM((nfeat, nhid), f32),
            pltpu.VMEM((nhid, nhid), jnp.bfloat16),
            pltpu.VMEM((nhid, ncls_p), jnp.bfloat16),
            pltpu.VMEM((1, nhid), f32),
            pltpu.VMEM((1, nhid), f32),
            pltpu.VMEM((n, n), jnp.bfloat16),
            pltpu.VMEM((n, nhid), jnp.bfloat16),
            pltpu.VMEM((n, nhid), jnp.bfloat16),
            pltpu.VMEM((n, nhid), jnp.bfloat16),
        ],
        compiler_params=pltpu.CompilerParams(
            dimension_semantics=("arbitrary",)),
    )(x, adj, w0, w1, w2p, v(b0), v(b1), b2p,
      v(g0), v(m0), v(v0), v(be0), v(g1), v(m1), v(v1), v(be1))
    return out


# speedup vs baseline: 1.0676x; 1.0016x over previous
import jax
import jax.numpy as jnp
from jax.experimental import pallas as pl
from jax.experimental.pallas import tpu as pltpu

NEG = -1e30
_ITERS = 5

_XT = 1024
_AT = 256
_RT = 1024

_N_X = 4
_N_A = 16
_N_R = 4

_S1 = 1 + _N_X
_L2 = _S1 + _N_A
_L3 = _L2 + _N_R
_STEPS = _L3 + _N_R


def _bjorck(w):
    for _ in range(_ITERS):
        wtw = jax.lax.dot_general(w, w, (((0,), (0,)), ((), ())),
                                  preferred_element_type=jnp.float32)
        w = 1.5 * w - 0.5 * jnp.dot(w, wtw,
                                    preferred_element_type=jnp.float32)
    return w


def _mega_kernel(x_ref, a_ref, w0_ref, w1_ref, w2_ref,
                 b0_ref, b1_ref, b2_ref, g0_ref, m0_ref, v0_ref, be0_ref,
                 g1_ref, m1_ref, v1_ref, be1_ref,
                 o_ref,
                 wf0_s, wf1_s, wf2_s, bf0_s, bf1_s, a_s, s1_s, h1_s, h2_s):
    s = pl.program_id(0)
    bf16 = jnp.bfloat16

    @pl.when(s == 0)
    def _init():
        sc0 = g0_ref[...] * jax.lax.rsqrt(v0_ref[...] + 1e-5)
        wf0_s[...] = _bjorck(w0_ref[...]) * sc0
        bf0_s[...] = (b0_ref[...] - m0_ref[...]) * sc0 + be0_ref[...]
        sc1 = g1_ref[...] * jax.lax.rsqrt(v1_ref[...] + 1e-5)
        bf1_s[...] = (b1_ref[...] - m1_ref[...]) * sc1 + be1_ref[...]

    @pl.when((s >= 1) & (s < _S1))
    def _sup1():
        r = (s - 1) * _XT
        s1_s[pl.ds(r, _XT), :] = jnp.dot(
            x_ref[...], wf0_s[...],
            preferred_element_type=jnp.float32).astype(bf16)

    @pl.when(s == _S1 + 1)
    def _bj1():
        sc1 = g1_ref[...] * jax.lax.rsqrt(v1_ref[...] + 1e-5)
        wf1_s[...] = (_bjorck(w1_ref[...]) * sc1).astype(bf16)

    @pl.when(s == _S1 + 2)
    def _bj2():
        wf2_s[...] = _bjorck(w2_ref[...]).astype(bf16)

    @pl.when((s >= _S1) & (s < _L2))
    def _stream():
        r = (s - _S1) * _AT
        a = a_ref[...].astype(bf16)
        a_s[pl.ds(r, _AT), :] = a
        h1_s[pl.ds(r, _AT), :] = jnp.maximum(
            jnp.dot(a, s1_s[...], preferred_element_type=jnp.float32)
            + bf0_s[...], 0.0).astype(bf16)

    @pl.when(s == _L2)
    def _sup2():
        s1_s[...] = jnp.dot(h1_s[...], wf1_s[...],
                            preferred_element_type=jnp.float32).astype(bf16)

    @pl.when((s >= _L2) & (s < _L3))
    def _layer2():
        r = (s - _L2) * _RT
        acc = jnp.dot(a_s[pl.ds(r, _RT), :], s1_s[...],
                      preferred_element_type=jnp.float32)
        h2_s[pl.ds(r, _RT), :] = jnp.maximum(
            acc + bf1_s[...], 0.0).astype(bf16)

    @pl.when(s == _L3)
    def _sup3():
        h1_s[:, :128] = jnp.dot(h2_s[...], wf2_s[...],
                                preferred_element_type=jnp.float32
                                ).astype(bf16)

    @pl.when(s >= _L3)
    def _layer3():
        r = (s - _L3) * _RT
        acc = jnp.dot(a_s[pl.ds(r, _RT), :], h1_s[:, :128],
                      preferred_element_type=jnp.float32)
        h = acc + b2_ref[...]
        m = jnp.max(h, axis=-1, keepdims=True)
        z = h - m
        lse = jnp.log(jnp.sum(jnp.exp(z), axis=-1, keepdims=True))
        o_ref[...] = (z - lse)[:, :64]


def kernel(x, adj, w0, b0, w1, b1, w2, b2,
           g0, be0, m0, v0, g1, be1, m1, v1):
    f32 = jnp.float32
    n = x.shape[0]
    nfeat, nhid = w0.shape
    nclass = w2.shape[1]
    ncls_p = 128

    b2p = jnp.pad(b2, (0, ncls_p - nclass),
                  constant_values=NEG).reshape(1, ncls_p).astype(f32)
    w2p = jnp.pad(w2, ((0, 0), (0, ncls_p - nclass)))
    v = lambda a: a.reshape(1, nhid)

    rst = lambda i: (0, 0)
    out = pl.pallas_call(
        _mega_kernel,
        out_shape=jax.ShapeDtypeStruct((n, nclass), f32),
        grid=(_STEPS,),
        in_specs=[
            pl.BlockSpec((_XT, nfeat),
                         lambda i: (jnp.clip(i - 1, 0, _N_X - 1), 0)),
            pl.BlockSpec((_AT, n),
                         lambda i: (jnp.clip(i - _S1, 0, _N_A - 1), 0)),
            pl.BlockSpec((nfeat, nhid), rst),
            pl.BlockSpec((nhid, nhid), rst),
            pl.BlockSpec((nhid, ncls_p), rst),
        ] + [pl.BlockSpec((1, nhid), rst)] * 2
          + [pl.BlockSpec((1, ncls_p), rst)]
          + [pl.BlockSpec((1, nhid), rst)] * 8,
        out_specs=pl.BlockSpec((_RT, nclass),
                               lambda i: (jnp.clip(i - _L3, 0, _N_R - 1), 0)),
        scratch_shapes=[
            pltpu.VMEM((nfeat, nhid), f32),
            pltpu.VMEM((nhid, nhid), jnp.bfloat16),
            pltpu.VMEM((nhid, ncls_p), jnp.bfloat16),
            pltpu.VMEM((1, nhid), f32),
            pltpu.VMEM((1, nhid), f32),
            pltpu.VMEM((n, n), jnp.bfloat16),
            pltpu.VMEM((n, nhid), jnp.bfloat16),
            pltpu.VMEM((n, nhid), jnp.bfloat16),
            pltpu.VMEM((n, nhid), jnp.bfloat16),
        ],
        compiler_params=pltpu.CompilerParams(
            dimension_semantics=("arbitrary",)),
    )(x, adj, w0, w1, w2p, v(b0), v(b1), b2p,
      v(g0), v(m0), v(v0), v(be0), v(g1), v(m1), v(v1), v(be1))
    return out
